# baseline (speedup 1.0000x reference)
import functools

import jax
import jax.numpy as jnp
from jax import lax
from jax.experimental import pallas as pl
from jax.experimental.pallas import tpu as pltpu
from jax.experimental.pallas import tpu_sc as plsc

R = 64
V = 1_000_000
B = 16384
NC = 2
NS = 16
RPC = R // NC
CPT = B // NS
H = V // 2
CH = 31248
TAIL = H - NS * CH


def _sc_gather(t, idx32):
    mesh = plsc.VectorSubcoreMesh(core_axis_name="c", subcore_axis_name="s")

    @functools.partial(
        pl.kernel,
        mesh=mesh,
        out_type=jax.ShapeDtypeStruct((R, B), jnp.float32),
        compiler_params=pltpu.CompilerParams(
            use_tc_tiling_on_sc=False,
            needs_layout_passes=False,
        ),
        scratch_types=[
            pltpu.VMEM((CPT,), jnp.int32),
            pltpu.VMEM((CPT,), jnp.int32),
            pltpu.VMEM((CPT,), jnp.int32),
            pltpu.VMEM((CPT,), jnp.float32),
            pltpu.VMEM((CPT,), jnp.float32),
            pltpu.VMEM((RPC, CPT), jnp.float32),
            pltpu.VMEM_SHARED((H,), jnp.float32),
            pltpu.VMEM_SHARED((H,), jnp.float32),
            pltpu.SemaphoreType.DMA,
            pltpu.SemaphoreType.DMA,
            pltpu.SemaphoreType.DMA,
        ],
    )
    def k(t_hbm, idx_hbm, out_hbm, idx_v, qa_v, qb_v, ga_v, gb_v, stage_v,
          s0_buf, s1_buf, sem0, sem1, semg):
        c = lax.axis_index("c")
        s = lax.axis_index("s")
        row0 = c * RPC
        cbase = s * CPT
        pltpu.sync_copy(idx_hbm.at[pl.ds(cbase, CPT)], idx_v)

        def qinit(i, _):
            sl = pl.ds(i * 16, 16)
            v = idx_v[sl]
            qa_v[sl] = jnp.minimum(v, H - 1)
            qb_v[sl] = jnp.maximum(v - H, 0)
            return 0

        lax.fori_loop(0, CPT // 16, qinit, 0)

        def stream_half(r, p, buf, sem):
            off = p * H
            pltpu.async_copy(
                t_hbm.at[row0 + r, pl.ds(off + s * CH, CH)],
                buf.at[pl.ds(s * CH, CH)],
                sem,
            )

            @pl.when(s == NS - 1)
            def _():
                pltpu.async_copy(
                    t_hbm.at[row0 + r, pl.ds(off + NS * CH, TAIL)],
                    buf.at[pl.ds(NS * CH, TAIL)],
                    sem,
                )

        def wait_stream(buf, sem):
            pltpu.make_async_copy(
                t_hbm.at[0, pl.ds(0, CH)], buf.at[pl.ds(0, CH)], sem
            ).wait()

            @pl.when(s == NS - 1)
            def _():
                pltpu.make_async_copy(
                    t_hbm.at[0, pl.ds(0, TAIL)], buf.at[pl.ds(0, TAIL)], sem
                ).wait()

        def row_body(r, _):
            wait_stream(s0_buf, sem0)
            plsc.subcore_barrier()
            pltpu.async_copy(s0_buf.at[qa_v], ga_v, semg).wait()
            plsc.subcore_barrier()

            @pl.when(r < RPC - 1)
            def _():
                stream_half(r + 1, 0, s0_buf, sem0)

            wait_stream(s1_buf, sem1)
            plsc.subcore_barrier()
            pltpu.async_copy(s1_buf.at[qb_v], gb_v, semg).wait()
            plsc.subcore_barrier()

            @pl.when(r < RPC - 1)
            def _():
                stream_half(r + 1, 1, s1_buf, sem1)

            def sel(i, _):
                sl = pl.ds(i * 16, 16)
                v = idx_v[sl]
                stage_v[r, sl] = jnp.where(v < H, ga_v[sl], gb_v[sl])
                return 0

            lax.fori_loop(0, CPT // 16, sel, 0)
            return 0

        stream_half(0, 0, s0_buf, sem0)
        stream_half(0, 1, s1_buf, sem1)
        lax.fori_loop(0, RPC, row_body, 0)
        pltpu.sync_copy(
            stage_v, out_hbm.at[pl.ds(row0, RPC), pl.ds(cbase, CPT)]
        )

    return k(t, idx32)


def kernel(t, idx):
    return _sc_gather(t, idx.astype(jnp.int32))


# speedup vs baseline: 15.8058x; 15.8058x over previous
import functools

import jax
import jax.numpy as jnp
from jax import lax
from jax.experimental import pallas as pl
from jax.experimental.pallas import tpu as pltpu
from jax.experimental.pallas import tpu_sc as plsc

R = 64
V = 1_000_000
B = 16384
NC = 2
NS = 16
NW = NC * NS
NB = 8
C = B // NW
NSLOT = 16


def _sc_gather_tiled(t8, idx32):
    mesh = plsc.VectorSubcoreMesh(core_axis_name="c", subcore_axis_name="s")

    @functools.partial(
        pl.kernel,
        mesh=mesh,
        out_type=jax.ShapeDtypeStruct((R * B,), jnp.float32),
        compiler_params=pltpu.CompilerParams(
            use_tc_tiling_on_sc=True, needs_layout_passes=False
        ),
        scratch_types=[
            pltpu.VMEM((C,), jnp.int32),
            pltpu.VMEM((NSLOT, NB, 128), jnp.float32),
            pltpu.VMEM((NB * NB * C,), jnp.float32),
            pltpu.SemaphoreType.DMA,
            pltpu.SemaphoreType.DMA,
            pltpu.SemaphoreType.DMA,
        ],
    )
    def k(t_hbm, idx_hbm, out_hbm, idx_v, ring_v, stage_v, semf0, semf1, semw):
        w = lax.axis_index("s") * NC + lax.axis_index("c")
        base = w * C
        pltpu.sync_copy(idx_hbm.at[pl.ds(base, C)], idx_v)

        lanes = lax.iota(jnp.int32, 16)
        rvec = lanes & 7
        hi_mask = lanes < 8

        def fetch(p, jt, sem):
            s0 = (p * NB) & (NSLOT - 1)
            for b in range(NB):
                pltpu.async_copy(
                    t_hbm.at[b, pl.ds(0, NB), pl.ds(jt * 128, 128)],
                    ring_v.at[s0 + b],
                    sem,
                )

        def drain_fetch(sem):
            for _ in range(NB):
                pltpu.make_async_copy(
                    t_hbm.at[0, pl.ds(0, NB), pl.ds(0, 128)],
                    ring_v.at[0],
                    sem,
                ).wait()

        def extract(p, jc):
            s0 = (p * NB) & (NSLOT - 1)
            cvec = jnp.full((16,), jc, jnp.int32)
            for m in range(4):
                bsel = (lanes >> 3) + 2 * m
                svec = (s0 + bsel) & (NSLOT - 1)
                vals = plsc.load_gather(ring_v, [svec, rvec, cvec])
                dest = (bsel * NB + rvec) * C + p
                plsc.store_scatter(stage_v, [dest], vals)

        def group(g, prev_jc):
            v = idx_v[pl.ds(g * 16, 16)]
            for kk in range(16):
                p = g * 16 + kk
                j = v[kk]
                sem = semf0 if kk % 2 == 0 else semf1
                osem = semf1 if kk % 2 == 0 else semf0
                fetch(p, j >> 7, sem)

                @pl.when(p > 0)
                def _():
                    drain_fetch(osem)
                    extract(p - 1, prev_jc)

                prev_jc = j & 127
            return prev_jc

        last_jc = lax.fori_loop(0, C // 16, group, jnp.int32(0))
        drain_fetch(semf1)
        extract(C - 1, last_jc)

        for i in range(R):
            pltpu.async_copy(
                stage_v.at[pl.ds(i * C, C)],
                out_hbm.at[pl.ds(i * B + base, C)],
                semw,
            )
        for i in range(R):
            pltpu.make_async_copy(
                stage_v.at[pl.ds(0, C)], out_hbm.at[pl.ds(0, C)], semw
            ).wait()

    return k(t8, idx32)


def kernel(t, idx):
    flat = _sc_gather_tiled(t.reshape(NB, NB, V), idx.astype(jnp.int32))
    return flat.reshape(R, B)
